# Initial kernel scaffold; baseline (speedup 1.0000x reference)
import jax
import jax.numpy as jnp
from jax.experimental import pallas as pl


def kernel(features_0, features_1, rbf, idnb_i, n_atoms, R, W_rbf, W1, b1, W2, b2, W3, b3, W_out, b_out, W_force):
    raise NotImplementedError("write your pallas kernel here")


# speedup vs baseline: 2.6881x; 2.6881x over previous
import functools

import jax
import jax.numpy as jnp
from jax import lax
from jax.experimental import pallas as pl
from jax.experimental.pallas import tpu as pltpu
from jax.experimental.pallas import tpu_sc as plsc

_E = 320000
_N = 10000
_EMB = 128
_OUT = 256
_NRBF = 16

_BE = 8000
_BN = 2000
_NROW = _E // 128
_RPT = _NROW // 16
_NPT = (_N // 16) // 8 * 8
_NTAIL = _N - 16 * _NPT


def _edge_body(rbf_ref, f0_ref, wrbf_ref, out_ref):
    g = jnp.dot(rbf_ref[...], wrbf_ref[...], preferred_element_type=jnp.float32)
    out_ref[...] = g * f0_ref[...]


def _edge_transform(rbf, features_0, W_rbf):
    return pl.pallas_call(
        _edge_body,
        grid=(_E // _BE,),
        in_specs=[
            pl.BlockSpec((_BE, _NRBF), lambda i: (i, 0)),
            pl.BlockSpec((_BE, _EMB), lambda i: (i, 0)),
            pl.BlockSpec((_NRBF, _EMB), lambda i: (0, 0)),
        ],
        out_specs=pl.BlockSpec((_BE, _EMB), lambda i: (i, 0)),
        out_shape=jax.ShapeDtypeStruct((_E, _EMB), jnp.float32),
    )(rbf, features_0, W_rbf)


def _seg_body(xs_hbm, f1_hbm, idx_hbm, z_hbm, out_hbm, idx_v, rows_v, acc_sh):
    c = lax.axis_index("c")
    s = lax.axis_index("s")

    pltpu.sync_copy(z_hbm, acc_sh.at[pl.ds(s * _NPT, _NPT)])

    @pl.when(s == 0)
    def _():
        pltpu.sync_copy(z_hbm.at[pl.ds(0, _NTAIL)],
                        acc_sh.at[pl.ds(16 * _NPT, _NTAIL)])

    plsc.subcore_barrier()

    def do_batch(r):
        pltpu.sync_copy(idx_hbm.at[r], idx_v)

        @pl.when(c == 0)
        def _():
            pltpu.sync_copy(xs_hbm.at[pl.ds(r * 128, 128)], rows_v)

        @pl.when(c == 1)
        def _():
            pltpu.sync_copy(f1_hbm.at[pl.ds(r * 128, 128)], rows_v)

        pltpu.sync_copy(rows_v, acc_sh.at[idx_v.at[0]], add=True)

    def body(i, carry):
        do_batch(s * _RPT + i)
        return carry

    lax.fori_loop(0, _RPT, body, 0)

    @pl.when(s < _NROW - 16 * _RPT)
    def _():
        do_batch(16 * _RPT + s)

    plsc.subcore_barrier()
    pltpu.sync_copy(acc_sh.at[pl.ds(s * _NPT, _NPT)],
                    out_hbm.at[c, pl.ds(s * _NPT, _NPT)])

    @pl.when(s == 0)
    def _():
        pltpu.sync_copy(acc_sh.at[pl.ds(16 * _NPT, _NTAIL)],
                        out_hbm.at[c, pl.ds(16 * _NPT, _NTAIL)])


def _segment_sums(x_scalar, features_1, idnb2, zeros):
    mesh = plsc.VectorSubcoreMesh(core_axis_name="c", subcore_axis_name="s")
    run = functools.partial(
        pl.kernel,
        out_type=jax.ShapeDtypeStruct((2, _N, _EMB), jnp.float32),
        mesh=mesh,
        scratch_types=[
            pltpu.VMEM((1, 128), jnp.int32),
            pltpu.VMEM((128, _EMB), jnp.float32),
            pltpu.VMEM_SHARED((_N, _EMB), jnp.float32),
        ],
    )(_seg_body)
    return run(x_scalar, features_1, idnb2, zeros)


def _mlp_body(x_ref, f_ref, w1_ref, b1_ref, w2_ref, b2_ref, w3_ref, b3_ref,
              wo_ref, bo_ref, wf_ref, e_ref, frc_ref):
    def swish(v):
        return v * jax.nn.sigmoid(v)

    x = x_ref[...]
    h = swish(jnp.dot(x, w1_ref[...], preferred_element_type=jnp.float32) + b1_ref[...])
    h = swish(jnp.dot(h, w2_ref[...], preferred_element_type=jnp.float32) + b2_ref[...])
    h = swish(jnp.dot(h, w3_ref[...], preferred_element_type=jnp.float32) + b3_ref[...])
    e_ref[...] = jnp.dot(h, wo_ref[...], preferred_element_type=jnp.float32) + bo_ref[...]
    frc_ref[...] = jnp.dot(f_ref[...], wf_ref[...], preferred_element_type=jnp.float32)


def _mlp(x_atoms, f_atoms, W1, b1, W2, b2, W3, b3, Wo_p, bo_p, Wf_p):
    full = lambda shape: pl.BlockSpec(shape, lambda i: (0, 0))
    return pl.pallas_call(
        _mlp_body,
        grid=(_N // _BN,),
        in_specs=[
            pl.BlockSpec((_BN, _EMB), lambda i: (i, 0)),
            pl.BlockSpec((_BN, _EMB), lambda i: (i, 0)),
            full((_EMB, _OUT)), full((1, _OUT)),
            full((_OUT, _OUT)), full((1, _OUT)),
            full((_OUT, _OUT)), full((1, _OUT)),
            full((_OUT, 128)), full((1, 128)),
            full((_EMB, 128)),
        ],
        out_specs=[
            pl.BlockSpec((_BN, 128), lambda i: (i, 0)),
            pl.BlockSpec((_BN, 128), lambda i: (i, 0)),
        ],
        out_shape=[
            jax.ShapeDtypeStruct((_N, 128), jnp.float32),
            jax.ShapeDtypeStruct((_N, 128), jnp.float32),
        ],
    )(x_atoms, f_atoms, W1, b1, W2, b2, W3, b3, Wo_p, bo_p, Wf_p)


def kernel(features_0, features_1, rbf, idnb_i, n_atoms, R,
           W_rbf, W1, b1, W2, b2, W3, b3, W_out, b_out, W_force):
    x_scalar = _edge_transform(rbf, features_0, W_rbf)

    idnb2 = jnp.reshape(idnb_i.astype(jnp.int32), (_NROW, 1, 128))
    zeros = jnp.zeros((_NPT, _EMB), jnp.float32)
    acc = _segment_sums(x_scalar, features_1, idnb2, zeros)

    Wo_p = jnp.pad(W_out, ((0, 0), (0, 128 - W_out.shape[1])))
    bo_p = jnp.pad(b_out, (0, 128 - b_out.shape[0]))[None, :]
    Wf_p = jnp.pad(W_force, ((0, 0), (0, 128 - W_force.shape[1])))
    e_pad, frc_pad = _mlp(acc[0], acc[1], W1, b1[None, :], W2, b2[None, :],
                          W3, b3[None, :], Wo_p, bo_p, Wf_p)
    energy = e_pad[:, :W_out.shape[1]]
    forces = frc_pad[:, :W_force.shape[1]]
    return (energy, forces)


# speedup vs baseline: 3.8780x; 1.4427x over previous
import functools

import jax
import jax.numpy as jnp
from jax import lax
from jax.experimental import pallas as pl
from jax.experimental.pallas import tpu as pltpu
from jax.experimental.pallas import tpu_sc as plsc

_E = 320000
_N = 10000
_EMB = 128
_OUT = 256
_NRBF = 16

_BE = 8000
_BN = 2000
_NROW = _E // 128
_RPT = _NROW // 16
_NPT = (_N // 16) // 8 * 8
_NTAIL = _N - 16 * _NPT


def _edge_body(rbf_ref, f0_ref, wrbf_ref, out_ref):
    g = jnp.dot(rbf_ref[...], wrbf_ref[...], preferred_element_type=jnp.float32)
    out_ref[...] = g * f0_ref[...]


def _edge_transform(rbf, features_0, W_rbf):
    return pl.pallas_call(
        _edge_body,
        grid=(_E // _BE,),
        in_specs=[
            pl.BlockSpec((_BE, _NRBF), lambda i: (i, 0)),
            pl.BlockSpec((_BE, _EMB), lambda i: (i, 0)),
            pl.BlockSpec((_NRBF, _EMB), lambda i: (0, 0)),
        ],
        out_specs=pl.BlockSpec((_BE, _EMB), lambda i: (i, 0)),
        out_shape=jax.ShapeDtypeStruct((_E, _EMB), jnp.float32),
    )(rbf, features_0, W_rbf)


def _seg_body(xs_hbm, f1_hbm, idx_hbm, z_hbm, out_hbm, idx_v, rows_v, acc_sh,
              isem, rsem):
    c = lax.axis_index("c")
    s = lax.axis_index("s")

    pltpu.sync_copy(z_hbm, acc_sh.at[pl.ds(s * _NPT, _NPT)])

    @pl.when(s == 0)
    def _():
        pltpu.sync_copy(z_hbm.at[pl.ds(0, _NTAIL)],
                        acc_sh.at[pl.ds(16 * _NPT, _NTAIL)])

    plsc.subcore_barrier()

    def start(r, b):
        pltpu.async_copy(idx_hbm.at[r], idx_v.at[b], isem.at[b])

        @pl.when(c == 0)
        def _():
            pltpu.async_copy(xs_hbm.at[pl.ds(r * 128, 128)], rows_v.at[b],
                             rsem.at[b])

        @pl.when(c == 1)
        def _():
            pltpu.async_copy(f1_hbm.at[pl.ds(r * 128, 128)], rows_v.at[b],
                             rsem.at[b])

    def finish(r, b):
        pltpu.make_async_copy(idx_hbm.at[r], idx_v.at[b], isem.at[b]).wait()
        pltpu.make_async_copy(xs_hbm.at[pl.ds(r * 128, 128)], rows_v.at[b],
                              rsem.at[b]).wait()
        pltpu.sync_copy(rows_v.at[b], acc_sh.at[idx_v.at[b, 0]], add=True)

    base = s * _RPT
    start(base, 0)

    def body(j, carry):
        i0 = base + 2 * j
        start(i0 + 1, 1)
        finish(i0, 0)

        @pl.when(2 * j + 2 < _RPT)
        def _():
            start(i0 + 2, 0)

        finish(i0 + 1, 1)
        return carry

    lax.fori_loop(0, _RPT // 2, body, 0)

    @pl.when(s < _NROW - 16 * _RPT)
    def _():
        r = 16 * _RPT + s
        start(r, 0)
        finish(r, 0)

    plsc.subcore_barrier()
    pltpu.sync_copy(acc_sh.at[pl.ds(s * _NPT, _NPT)],
                    out_hbm.at[c, pl.ds(s * _NPT, _NPT)])

    @pl.when(s == 0)
    def _():
        pltpu.sync_copy(acc_sh.at[pl.ds(16 * _NPT, _NTAIL)],
                        out_hbm.at[c, pl.ds(16 * _NPT, _NTAIL)])


def _segment_sums(x_scalar, features_1, idnb2, zeros):
    mesh = plsc.VectorSubcoreMesh(core_axis_name="c", subcore_axis_name="s")
    run = functools.partial(
        pl.kernel,
        out_type=jax.ShapeDtypeStruct((2, _N, _EMB), jnp.float32),
        mesh=mesh,
        scratch_types=[
            pltpu.VMEM((2, 1, 128), jnp.int32),
            pltpu.VMEM((2, 128, _EMB), jnp.float32),
            pltpu.VMEM_SHARED((_N, _EMB), jnp.float32),
            pltpu.SemaphoreType.DMA((2,)),
            pltpu.SemaphoreType.DMA((2,)),
        ],
    )(_seg_body)
    return run(x_scalar, features_1, idnb2, zeros)


def _mlp_body(x_ref, f_ref, w1_ref, b1_ref, w2_ref, b2_ref, w3_ref, b3_ref,
              wo_ref, bo_ref, wf_ref, e_ref, frc_ref):
    def swish(v):
        return v * jax.nn.sigmoid(v)

    x = x_ref[...]
    h = swish(jnp.dot(x, w1_ref[...], preferred_element_type=jnp.float32) + b1_ref[...])
    h = swish(jnp.dot(h, w2_ref[...], preferred_element_type=jnp.float32) + b2_ref[...])
    h = swish(jnp.dot(h, w3_ref[...], preferred_element_type=jnp.float32) + b3_ref[...])
    e_ref[...] = jnp.dot(h, wo_ref[...], preferred_element_type=jnp.float32) + bo_ref[...]
    frc_ref[...] = jnp.dot(f_ref[...], wf_ref[...], preferred_element_type=jnp.float32)


def _mlp(x_atoms, f_atoms, W1, b1, W2, b2, W3, b3, Wo_p, bo_p, Wf_p):
    full = lambda shape: pl.BlockSpec(shape, lambda i: (0, 0))
    return pl.pallas_call(
        _mlp_body,
        grid=(_N // _BN,),
        in_specs=[
            pl.BlockSpec((_BN, _EMB), lambda i: (i, 0)),
            pl.BlockSpec((_BN, _EMB), lambda i: (i, 0)),
            full((_EMB, _OUT)), full((1, _OUT)),
            full((_OUT, _OUT)), full((1, _OUT)),
            full((_OUT, _OUT)), full((1, _OUT)),
            full((_OUT, 128)), full((1, 128)),
            full((_EMB, 128)),
        ],
        out_specs=[
            pl.BlockSpec((_BN, 128), lambda i: (i, 0)),
            pl.BlockSpec((_BN, 128), lambda i: (i, 0)),
        ],
        out_shape=[
            jax.ShapeDtypeStruct((_N, 128), jnp.float32),
            jax.ShapeDtypeStruct((_N, 128), jnp.float32),
        ],
    )(x_atoms, f_atoms, W1, b1, W2, b2, W3, b3, Wo_p, bo_p, Wf_p)


def kernel(features_0, features_1, rbf, idnb_i, n_atoms, R,
           W_rbf, W1, b1, W2, b2, W3, b3, W_out, b_out, W_force):
    x_scalar = _edge_transform(rbf, features_0, W_rbf)

    idnb2 = jnp.reshape(idnb_i.astype(jnp.int32), (_NROW, 1, 128))
    zeros = jnp.zeros((_NPT, _EMB), jnp.float32)
    acc = _segment_sums(x_scalar, features_1, idnb2, zeros)

    Wo_p = jnp.pad(W_out, ((0, 0), (0, 128 - W_out.shape[1])))
    bo_p = jnp.pad(b_out, (0, 128 - b_out.shape[0]))[None, :]
    Wf_p = jnp.pad(W_force, ((0, 0), (0, 128 - W_force.shape[1])))
    e_pad, frc_pad = _mlp(acc[0], acc[1], W1, b1[None, :], W2, b2[None, :],
                          W3, b3[None, :], Wo_p, bo_p, Wf_p)
    energy = e_pad[:, :W_out.shape[1]]
    forces = frc_pad[:, :W_force.shape[1]]
    return (energy, forces)


# speedup vs baseline: 5.1751x; 1.3345x over previous
import functools

import jax
import jax.numpy as jnp
from jax import lax
from jax.experimental import pallas as pl
from jax.experimental.pallas import tpu as pltpu
from jax.experimental.pallas import tpu_sc as plsc

_E = 320000
_N = 10000
_EMB = 128
_OUT = 256
_NRBF = 16

_BE = 16000
_BN = 2000
_NROW = _E // 128
_RPT = _NROW // 16
_NPT = (_N // 16) // 8 * 8
_NTAIL = _N - 16 * _NPT


def _edge_body(rbft_ref, f0_ref, wrbf_ref, out_ref):
    g = lax.dot_general(rbft_ref[...], wrbf_ref[...],
                        (((0,), (0,)), ((), ())),
                        preferred_element_type=jnp.float32)
    out_ref[...] = g * f0_ref[...]


def _edge_transform(rbf_t, features_0, W_rbf):
    return pl.pallas_call(
        _edge_body,
        grid=(_E // _BE,),
        in_specs=[
            pl.BlockSpec((_NRBF, _BE), lambda i: (0, i)),
            pl.BlockSpec((_BE, _EMB), lambda i: (i, 0)),
            pl.BlockSpec((_NRBF, _EMB), lambda i: (0, 0)),
        ],
        out_specs=pl.BlockSpec((_BE, _EMB), lambda i: (i, 0)),
        out_shape=jax.ShapeDtypeStruct((_E, _EMB), jnp.float32),
    )(rbf_t, features_0, W_rbf)


def _seg_body(xs_hbm, f1_hbm, idx_hbm, z_hbm, out_hbm, idx_v, rows_v, acc_sh,
              isem, rsem):
    c = lax.axis_index("c")
    s = lax.axis_index("s")

    pltpu.sync_copy(z_hbm, acc_sh.at[pl.ds(s * _NPT, _NPT)])

    @pl.when(s == 0)
    def _():
        pltpu.sync_copy(z_hbm.at[pl.ds(0, _NTAIL)],
                        acc_sh.at[pl.ds(16 * _NPT, _NTAIL)])

    plsc.subcore_barrier()

    def start(r, b):
        pltpu.async_copy(idx_hbm.at[r], idx_v.at[b], isem.at[b])

        @pl.when(c == 0)
        def _():
            pltpu.async_copy(xs_hbm.at[pl.ds(r * 128, 128)], rows_v.at[b],
                             rsem.at[b])

        @pl.when(c == 1)
        def _():
            pltpu.async_copy(f1_hbm.at[pl.ds(r * 128, 128)], rows_v.at[b],
                             rsem.at[b])

    def finish(r, b):
        pltpu.make_async_copy(idx_hbm.at[r], idx_v.at[b], isem.at[b]).wait()
        pltpu.make_async_copy(xs_hbm.at[pl.ds(r * 128, 128)], rows_v.at[b],
                              rsem.at[b]).wait()
        pltpu.sync_copy(rows_v.at[b], acc_sh.at[idx_v.at[b, 0]], add=True)

    base = s * _RPT
    start(base, 0)

    def body(j, carry):
        i0 = base + 2 * j
        start(i0 + 1, 1)
        finish(i0, 0)

        @pl.when(2 * j + 2 < _RPT)
        def _():
            start(i0 + 2, 0)

        finish(i0 + 1, 1)
        return carry

    lax.fori_loop(0, _RPT // 2, body, 0)

    @pl.when(s < _NROW - 16 * _RPT)
    def _():
        r = 16 * _RPT + s
        start(r, 0)
        finish(r, 0)

    plsc.subcore_barrier()
    pltpu.sync_copy(acc_sh.at[pl.ds(s * _NPT, _NPT)],
                    out_hbm.at[c, pl.ds(s * _NPT, _NPT)])

    @pl.when(s == 0)
    def _():
        pltpu.sync_copy(acc_sh.at[pl.ds(16 * _NPT, _NTAIL)],
                        out_hbm.at[c, pl.ds(16 * _NPT, _NTAIL)])


def _segment_sums(x_scalar, features_1, idnb2, zeros):
    mesh = plsc.VectorSubcoreMesh(core_axis_name="c", subcore_axis_name="s")
    run = functools.partial(
        pl.kernel,
        out_type=jax.ShapeDtypeStruct((2, _N, _EMB), jnp.float32),
        mesh=mesh,
        scratch_types=[
            pltpu.VMEM((2, 1, 128), jnp.int32),
            pltpu.VMEM((2, 128, _EMB), jnp.float32),
            pltpu.VMEM_SHARED((_N, _EMB), jnp.float32),
            pltpu.SemaphoreType.DMA((2,)),
            pltpu.SemaphoreType.DMA((2,)),
        ],
    )(_seg_body)
    return run(x_scalar, features_1, idnb2, zeros)


def _mlp_body(x_ref, f_ref, w1_ref, b1_ref, w2_ref, b2_ref, w3_ref, b3_ref,
              wo_ref, bo_ref, wf_ref, e_ref, frc_ref):
    def swish(v):
        return v * jax.nn.sigmoid(v)

    x = x_ref[...]
    h = swish(jnp.dot(x, w1_ref[...], preferred_element_type=jnp.float32) + b1_ref[...])
    h = swish(jnp.dot(h, w2_ref[...], preferred_element_type=jnp.float32) + b2_ref[...])
    h = swish(jnp.dot(h, w3_ref[...], preferred_element_type=jnp.float32) + b3_ref[...])
    e_ref[...] = jnp.dot(h, wo_ref[...], preferred_element_type=jnp.float32) + bo_ref[...]
    frc_ref[...] = jnp.dot(f_ref[...], wf_ref[...], preferred_element_type=jnp.float32)


def _mlp(x_atoms, f_atoms, W1, b1, W2, b2, W3, b3, Wo_p, bo_p, Wf_p):
    full = lambda shape: pl.BlockSpec(shape, lambda i: (0, 0))
    return pl.pallas_call(
        _mlp_body,
        grid=(_N // _BN,),
        in_specs=[
            pl.BlockSpec((_BN, _EMB), lambda i: (i, 0)),
            pl.BlockSpec((_BN, _EMB), lambda i: (i, 0)),
            full((_EMB, _OUT)), full((1, _OUT)),
            full((_OUT, _OUT)), full((1, _OUT)),
            full((_OUT, _OUT)), full((1, _OUT)),
            full((_OUT, 128)), full((1, 128)),
            full((_EMB, 128)),
        ],
        out_specs=[
            pl.BlockSpec((_BN, 128), lambda i: (i, 0)),
            pl.BlockSpec((_BN, 128), lambda i: (i, 0)),
        ],
        out_shape=[
            jax.ShapeDtypeStruct((_N, 128), jnp.float32),
            jax.ShapeDtypeStruct((_N, 128), jnp.float32),
        ],
    )(x_atoms, f_atoms, W1, b1, W2, b2, W3, b3, Wo_p, bo_p, Wf_p)


def kernel(features_0, features_1, rbf, idnb_i, n_atoms, R,
           W_rbf, W1, b1, W2, b2, W3, b3, W_out, b_out, W_force):
    x_scalar = _edge_transform(rbf.T, features_0, W_rbf)

    idnb2 = jnp.reshape(idnb_i.astype(jnp.int32), (_NROW, 1, 128))
    zeros = jnp.zeros((_NPT, _EMB), jnp.float32)
    acc = _segment_sums(x_scalar, features_1, idnb2, zeros)

    Wo_p = jnp.pad(W_out, ((0, 0), (0, 128 - W_out.shape[1])))
    bo_p = jnp.pad(b_out, (0, 128 - b_out.shape[0]))[None, :]
    Wf_p = jnp.pad(W_force, ((0, 0), (0, 128 - W_force.shape[1])))
    e_pad, frc_pad = _mlp(acc[0], acc[1], W1, b1[None, :], W2, b2[None, :],
                          W3, b3[None, :], Wo_p, bo_p, Wf_p)
    energy = e_pad[:, :W_out.shape[1]]
    forces = frc_pad[:, :W_force.shape[1]]
    return (energy, forces)
